# baseline (speedup 1.0000x reference)
import jax
import jax.numpy as jnp
from jax.experimental import pallas as pl

_N = 512


def _pipeline_kernel(att_ref, x_ref, a_ref, w1_ref, b1_ref, w2_ref, b2_ref,
                     w3_ref, b3_ref, wg1_ref, bg1_ref, wg2_ref, bg2_ref,
                     wg3_ref, bg3_ref, wg4_ref, bg4_ref, wg5_ref, bg5_ref,
                     wg6_ref, bg6_ref, out_ref):
    f32 = jnp.float32

    att = att_ref[...]
    e = jnp.exp(att - jnp.max(att))
    scores = e / jnp.sum(e)

    def dot(p, q):
        return jax.lax.dot_general(p, q, (((1,), (0,)), ((), ())),
                                   preferred_element_type=f32)

    def rowb(b_ref):
        return b_ref[...].reshape(1, -1)

    x = x_ref[...]
    mask = (a_ref[...] != 0.0).astype(f32)
    ones_row = jnp.ones((1, _N), f32)

    def colsum(p):
        return jax.lax.dot_general(ones_row, p, (((1,), (0,)), ((), ())),
                                   preferred_element_type=f32)

    deg = colsum(mask) + 1.0
    dinv = jax.lax.rsqrt(deg).reshape(_N, 1)

    x1l = dot(x, w1_ref[...]) + rowb(b1_ref)
    t = dinv * dot(x1l, wg1_ref[...])
    mt = jax.lax.dot_general(mask, t, (((0,), (0,)), ((), ())),
                             preferred_element_type=f32)
    x1 = x1l + jnp.maximum(dinv * (mt + t) + rowb(bg1_ref), 0.0)

    c2 = 1.0 / jnp.float32(_N + 1)

    def agg(h, wg_ref):
        p = dot(h, c2 * wg_ref[...])
        return colsum(p) + p

    x2l = dot(x1, w2_ref[...]) + rowb(b2_ref)
    x2 = x2l + jnp.maximum(agg(x2l, wg2_ref) + rowb(bg2_ref), 0.0)
    x3l = dot(x2, w3_ref[...]) + rowb(b3_ref)
    x3 = jnp.maximum(agg(x3l, wg3_ref) + rowb(bg3_ref), 0.0)
    x4 = jnp.maximum(agg(x3, wg4_ref) + rowb(bg4_ref), 0.0)
    x5 = jnp.maximum(agg(x4, wg5_ref) + rowb(bg5_ref), 0.0)
    x6 = agg(x5, wg6_ref) + rowb(bg6_ref)

    out_ref[...] = (x3l * scores[0] + x3 * scores[1] + x4 * scores[2] +
                    x5 * scores[3] + x6 * scores[4])


def kernel(x, A_neg, A_pos, W1, b1, W2, b2, W3, b3, Wg1, bg1, Wg2, bg2,
           Wg3, bg3, Wg4, bg4, Wg5, bg5, Wg6, bg6, att):
    return pl.pallas_call(
        _pipeline_kernel,
        out_shape=jax.ShapeDtypeStruct((_N, 64), jnp.float32),
    )(att, x, A_neg, W1, b1, W2, b2, W3, b3, Wg1, bg1, Wg2, bg2,
      Wg3, bg3, Wg4, bg4, Wg5, bg5, Wg6, bg6)


# speedup vs baseline: 1.0741x; 1.0741x over previous
import jax
import jax.numpy as jnp
from jax.experimental import pallas as pl

_N = 512


def _pipeline_kernel(att_ref, x_ref, a_ref, w1_ref, b1_ref, w2_ref, b2_ref,
                     w3_ref, b3_ref, wg1_ref, bg1_ref, wg2_ref, bg2_ref,
                     wg3_ref, bg3_ref, wg4_ref, bg4_ref, wg5_ref, bg5_ref,
                     wg6_ref, bg6_ref, out_ref):
    f32 = jnp.float32

    att = att_ref[...]
    e = jnp.exp(att - jnp.max(att))
    scores = e / jnp.sum(e)

    def dot(p, q):
        return jax.lax.dot_general(p, q, (((1,), (0,)), ((), ())),
                                   preferred_element_type=f32)

    def rowb(b_ref):
        return b_ref[...].reshape(1, -1)

    x = x_ref[...]
    mask = (a_ref[...] != 0.0).astype(f32)
    deg = jnp.sum(mask, axis=0, keepdims=True) + 1.0
    dinv = jax.lax.rsqrt(deg).reshape(_N, 1)

    x1l = dot(x, w1_ref[...]) + rowb(b1_ref)
    t = dinv * dot(x1l, wg1_ref[...])
    mt = jax.lax.dot_general(mask, t, (((0,), (0,)), ((), ())),
                             preferred_element_type=f32)
    x1 = x1l + jnp.maximum(dinv * (mt + t) + rowb(bg1_ref), 0.0)

    c2 = 1.0 / jnp.float32(_N + 1)

    def agg(h, wg_ref):
        p = dot(h, c2 * wg_ref[...])
        return jnp.sum(p, axis=0, keepdims=True) + p

    x2l = dot(x1, w2_ref[...]) + rowb(b2_ref)
    x2 = x2l + jnp.maximum(agg(x2l, wg2_ref) + rowb(bg2_ref), 0.0)
    x3l = dot(x2, w3_ref[...]) + rowb(b3_ref)
    x3 = jnp.maximum(agg(x3l, wg3_ref) + rowb(bg3_ref), 0.0)
    x4 = jnp.maximum(agg(x3, wg4_ref) + rowb(bg4_ref), 0.0)
    x5 = jnp.maximum(agg(x4, wg5_ref) + rowb(bg5_ref), 0.0)
    x6 = agg(x5, wg6_ref) + rowb(bg6_ref)

    out_ref[...] = (x3l * scores[0] + x3 * scores[1] + x4 * scores[2] +
                    x5 * scores[3] + x6 * scores[4])


def kernel(x, A_neg, A_pos, W1, b1, W2, b2, W3, b3, Wg1, bg1, Wg2, bg2,
           Wg3, bg3, Wg4, bg4, Wg5, bg5, Wg6, bg6, att):
    return pl.pallas_call(
        _pipeline_kernel,
        out_shape=jax.ShapeDtypeStruct((_N, 64), jnp.float32),
    )(att, x, A_neg, W1, b1, W2, b2, W3, b3, Wg1, bg1, Wg2, bg2,
      Wg3, bg3, Wg4, bg4, Wg5, bg5, Wg6, bg6)


# speedup vs baseline: 1.0825x; 1.0078x over previous
import jax
import jax.numpy as jnp
from jax.experimental import pallas as pl

_N = 512


def _pipeline_kernel(att_ref, x_ref, a_ref, w1_ref, b1_ref, w2_ref, b2_ref,
                     w3_ref, b3_ref, wg1_ref, bg1_ref, wg2_ref, bg2_ref,
                     wg3_ref, bg3_ref, wg4_ref, bg4_ref, wg5_ref, bg5_ref,
                     wg6_ref, bg6_ref, out_ref):
    f32 = jnp.float32

    att = att_ref[...]
    e = jnp.exp(att - jnp.max(att))
    scores = e / jnp.sum(e)

    def dot(p, q):
        return jax.lax.dot_general(p, q, (((1,), (0,)), ((), ())),
                                   preferred_element_type=f32)

    def rowb(b_ref):
        return b_ref[...].reshape(1, -1)

    x = x_ref[...]
    mask = (a_ref[...] != 0.0).astype(f32)
    deg = jnp.sum(mask, axis=0, keepdims=True) + 1.0
    dinv = jax.lax.rsqrt(deg).reshape(_N, 1)

    bf16 = jnp.bfloat16

    def dotb(p, q):
        return jax.lax.dot_general(p.astype(bf16), q.astype(bf16),
                                   (((1,), (0,)), ((), ())),
                                   preferred_element_type=f32)

    x1l = dotb(x, w1_ref[...]) + rowb(b1_ref)
    t = dinv * dotb(x1l, wg1_ref[...])
    mt = jax.lax.dot_general(mask.astype(bf16), t.astype(bf16),
                             (((0,), (0,)), ((), ())),
                             preferred_element_type=f32)
    x1 = x1l + jnp.maximum(dinv * (mt + t) + rowb(bg1_ref), 0.0)

    c2 = 1.0 / jnp.float32(_N + 1)

    def agg(h, wg_ref):
        p = dot(h, c2 * wg_ref[...])
        return jnp.sum(p, axis=0, keepdims=True) + p

    x2l = dot(x1, w2_ref[...]) + rowb(b2_ref)
    x2 = x2l + jnp.maximum(agg(x2l, wg2_ref) + rowb(bg2_ref), 0.0)
    x3l = dot(x2, w3_ref[...]) + rowb(b3_ref)
    x3 = jnp.maximum(agg(x3l, wg3_ref) + rowb(bg3_ref), 0.0)
    x4 = jnp.maximum(agg(x3, wg4_ref) + rowb(bg4_ref), 0.0)
    x5 = jnp.maximum(agg(x4, wg5_ref) + rowb(bg5_ref), 0.0)
    x6 = agg(x5, wg6_ref) + rowb(bg6_ref)

    out_ref[...] = (x3l * scores[0] + x3 * scores[1] + x4 * scores[2] +
                    x5 * scores[3] + x6 * scores[4])


def kernel(x, A_neg, A_pos, W1, b1, W2, b2, W3, b3, Wg1, bg1, Wg2, bg2,
           Wg3, bg3, Wg4, bg4, Wg5, bg5, Wg6, bg6, att):
    return pl.pallas_call(
        _pipeline_kernel,
        out_shape=jax.ShapeDtypeStruct((_N, 64), jnp.float32),
    )(att, x, A_neg, W1, b1, W2, b2, W3, b3, Wg1, bg1, Wg2, bg2,
      Wg3, bg3, Wg4, bg4, Wg5, bg5, Wg6, bg6)
